# baseline (speedup 1.0000x reference)
import jax
import jax.numpy as jnp
from jax.experimental import pallas as pl
from jax.experimental.pallas import tpu as pltpu

_NCH = 4


def _mlp_body(x_hbm, eps_ref, w1_ref, b1_ref, w2_ref, b2_ref, o_hbm,
              x_buf, o_buf, w1_bf, w2_bf, in_sems, out_sems):
    B = x_hbm.shape[0]
    cm = B // _NCH

    def in_copy(ci, slot):
        return pltpu.make_async_copy(
            x_hbm.at[pl.ds(ci * cm, cm)], x_buf.at[slot], in_sems.at[slot])

    def out_copy(ci, slot):
        return pltpu.make_async_copy(
            o_buf.at[slot], o_hbm.at[pl.ds(ci * cm, cm)], out_sems.at[slot])

    in_copy(0, 0).start()

    w1_bf[...] = w1_ref[...].astype(jnp.bfloat16)
    w2_bf[...] = w2_ref[...].astype(jnp.bfloat16)

    for ci in range(_NCH):
        slot = ci % 2
        if ci + 1 < _NCH:
            in_copy(ci + 1, (ci + 1) % 2).start()
        in_copy(ci, slot).wait()
        xe = jnp.concatenate(
            [x_buf[slot].astype(jnp.bfloat16),
             eps_ref[pl.ds(ci * cm, cm), :].astype(jnp.bfloat16)],
            axis=1)
        h = jnp.dot(xe, w1_bf[...], preferred_element_type=jnp.float32)
        h = jnp.maximum(h + b1_ref[...], 0.0).astype(jnp.bfloat16)
        o = jnp.dot(h, w2_bf[...], preferred_element_type=jnp.float32)
        o = o + b2_ref[...]
        if ci >= 2:
            out_copy(ci - 2, slot).wait()
        o_buf[slot] = jnp.maximum(o, 0.0) + jnp.log1p(jnp.exp(-jnp.abs(o)))
        out_copy(ci, slot).start()

    out_copy(_NCH - 2, (_NCH - 2) % 2).wait()
    out_copy(_NCH - 1, (_NCH - 1) % 2).wait()


def kernel(x, eps, W1, b1, W2, b2):
    B, d_in = x.shape
    noise_dim = eps.shape[1]
    H = W1.shape[1]
    d_out = W2.shape[1]
    cm = B // _NCH

    b1r = b1.reshape(1, H)
    b2r = b2.reshape(1, d_out)

    return pl.pallas_call(
        _mlp_body,
        in_specs=[
            pl.BlockSpec(memory_space=pl.ANY),
            pl.BlockSpec((B, noise_dim), lambda: (0, 0)),
            pl.BlockSpec((d_in + noise_dim, H), lambda: (0, 0)),
            pl.BlockSpec((1, H), lambda: (0, 0)),
            pl.BlockSpec((H, d_out), lambda: (0, 0)),
            pl.BlockSpec((1, d_out), lambda: (0, 0)),
        ],
        out_specs=pl.BlockSpec(memory_space=pl.ANY),
        out_shape=jax.ShapeDtypeStruct((B, d_out), jnp.float32),
        scratch_shapes=[
            pltpu.VMEM((2, cm, d_in), jnp.float32),
            pltpu.VMEM((2, cm, d_out), jnp.float32),
            pltpu.VMEM((d_in + noise_dim, H), jnp.bfloat16),
            pltpu.VMEM((H, d_out), jnp.bfloat16),
            pltpu.SemaphoreType.DMA((2,)),
            pltpu.SemaphoreType.DMA((2,)),
        ],
    )(x, eps, W1, b1r, W2, b2r)
